# baseline (speedup 1.0000x reference)
import jax
import jax.numpy as jnp
from jax.experimental import pallas as pl
from jax.experimental.pallas import tpu as pltpu

_DN_T = (((1,), (1,)), ((), ()))

_CH = 4096


def _pipeline(x_hbm, w1_ref, b1_ref, w2_ref, b2_ref, w3_ref, b3_ref,
              wc_ref, o_hbm, ov, xv, sem_in, sem_o):
    n, d = x_hbm.shape
    n_chunks = n // _CH

    def _in_copy(ci):
        return pltpu.make_async_copy(
            x_hbm.at[pl.ds(ci * _CH, _CH), :], xv.at[ci], sem_in.at[ci])

    def _o_copy(ci):
        return pltpu.make_async_copy(
            ov.at[ci], o_hbm.at[pl.ds(ci * _CH, _CH), :], sem_o.at[ci])

    for ci in range(n_chunks):
        _in_copy(ci).start()

    def step(ci, carry):
        _in_copy(ci).wait()
        h = jax.lax.dot_general(xv[ci], w1_ref[...], _DN_T,
                                preferred_element_type=jnp.float32)
        h = jnp.maximum(h + b1_ref[...], 0.0)
        h = jax.lax.dot_general(h, w2_ref[...], _DN_T,
                                preferred_element_type=jnp.float32)
        h = jnp.maximum(h + b2_ref[...], 0.0)
        f = jax.lax.dot_general(h, w3_ref[...], _DN_T,
                                preferred_element_type=jnp.float32)
        f = jnp.maximum(f + b3_ref[...], 0.0)
        s = jax.lax.dot_general(f, wc_ref[...], _DN_T,
                                preferred_element_type=jnp.float32)
        ov[ci] = jnp.concatenate([f, s], axis=1)
        _o_copy(ci).start()
        return carry

    jax.lax.fori_loop(0, n_chunks, step, 0, unroll=True)

    for ci in range(n_chunks):
        _o_copy(ci).wait()


def kernel(x, W1, b1, W2, b2, W3, b3, Wc):
    N, D = x.shape
    H1 = W1.shape[0]
    H2 = W2.shape[0]
    H3 = W3.shape[0]
    C = Wc.shape[0]
    n_chunks = N // _CH

    hbm = pl.BlockSpec(memory_space=pltpu.MemorySpace.HBM)
    vmem = pl.BlockSpec(memory_space=pltpu.MemorySpace.VMEM)

    out = pl.pallas_call(
        _pipeline,
        in_specs=[hbm, vmem, vmem, vmem, vmem, vmem, vmem, vmem],
        out_specs=hbm,
        out_shape=jax.ShapeDtypeStruct((N, H3 + C), jnp.float32),
        scratch_shapes=[
            pltpu.VMEM((n_chunks, _CH, H3 + C), jnp.float32),
            pltpu.VMEM((n_chunks, _CH, D), jnp.float32),
            pltpu.SemaphoreType.DMA((n_chunks,)),
            pltpu.SemaphoreType.DMA((n_chunks,)),
        ],
    )(x, W1, b1, W2, b2, W3, b3, Wc)
    return (out[:, :H3], out[:, H3:H3 + C])
